# Initial kernel scaffold; baseline (speedup 1.0000x reference)
import functools

import jax
import jax.numpy as jnp
from jax import lax
from jax.experimental import pallas as pl
from jax.experimental.pallas import tpu as pltpu
from jax.experimental.pallas import tpu_sc as plsc

N_SRC = 100000
N_DST = 100000
D = 128
E = 600000

NC = 2
NS = 16
NW = NC * NS
L = 16

CHUNK = 256
N_CHUNKS = -(-E // (NW * CHUNK))
PER_W = N_CHUNKS * CHUNK
EP = NW * PER_W
UNROLL = 4


def _body(xs_hbm, xd_hbm, i0_hbm, i1_hbm, out_hbm,
          i0_v, i1_v, rs_v, rd_v, o_v, sem0, sem1):
    cid = lax.axis_index("c")
    sid = lax.axis_index("s")
    wid = sid * NC + cid

    def chunk_body(k, carry):
        base = wid * PER_W + k * CHUNK
        pltpu.sync_copy(i0_hbm.at[pl.ds(base, CHUNK)], i0_v)
        pltpu.sync_copy(i1_hbm.at[pl.ds(base, CHUNK)], i1_v)
        cp0 = pltpu.async_copy(xs_hbm.at[i0_v], rs_v, sem0)
        cp1 = pltpu.async_copy(xd_hbm.at[i1_v], rd_v, sem1)
        cp0.wait()
        cp1.wait()

        def group_body(g, gcarry):
            e0 = g * L
            eids = lax.iota(jnp.int32, L) + e0
            zero = jnp.zeros((L,), jnp.float32)

            def col_body(cc, accs):
                col = cc * UNROLL
                new = []
                for u in range(UNROLL):
                    cs = jnp.full((L,), col + u, jnp.int32)
                    vs = plsc.load_gather(rs_v, [eids, cs])
                    vd = plsc.load_gather(rd_v, [eids, cs])
                    new.append(accs[u] + vs * vd)
                return tuple(new)

            accs = lax.fori_loop(0, D // UNROLL, col_body,
                                 (zero,) * UNROLL)
            o_v[pl.ds(e0, L)] = (accs[0] + accs[1]) + (accs[2] + accs[3])
            return gcarry

        lax.fori_loop(0, CHUNK // L, group_body, 0)
        pltpu.sync_copy(o_v, out_hbm.at[pl.ds(base, CHUNK)])
        return carry

    lax.fori_loop(0, N_CHUNKS, chunk_body, 0)


@jax.jit
def _run(x_src, x_dst, i0, i1):
    mesh = plsc.VectorSubcoreMesh(core_axis_name="c", subcore_axis_name="s")
    f = pl.kernel(
        _body,
        out_type=jax.ShapeDtypeStruct((EP,), jnp.float32),
        mesh=mesh,
        scratch_types=[
            pltpu.VMEM((CHUNK,), jnp.int32),
            pltpu.VMEM((CHUNK,), jnp.int32),
            pltpu.VMEM((CHUNK, D), jnp.float32),
            pltpu.VMEM((CHUNK, D), jnp.float32),
            pltpu.VMEM((CHUNK,), jnp.float32),
            pltpu.SemaphoreType.DMA,
            pltpu.SemaphoreType.DMA,
        ],
    )
    return f(x_src, x_dst, i0, i1)


def kernel(x_src, x_dst, edge_label_index):
    pad = EP - E
    idx = jnp.pad(edge_label_index, ((0, 0), (0, pad)))
    out = _run(x_src, x_dst, idx[0], idx[1])
    return out[:E]


# speedup vs baseline: 3.1804x; 3.1804x over previous
import functools

import jax
import jax.numpy as jnp
from jax import lax
from jax.experimental import pallas as pl
from jax.experimental.pallas import tpu as pltpu
from jax.experimental.pallas import tpu_sc as plsc

N_SRC = 100000
N_DST = 100000
D = 128
E = 600000

NC = 2
NS = 16
NW = NC * NS
L = 16

CHUNK = 256
N_CHUNKS = -(-E // (NW * CHUNK))
PER_W = N_CHUNKS * CHUNK
EP = NW * PER_W
UNROLL = 4


def _body(xs_hbm, xd_hbm, i0_hbm, i1_hbm, out_hbm,
          i0_v, i1_v, rs_v, rd_v, o_v, sem0, sem1):
    cid = lax.axis_index("c")
    sid = lax.axis_index("s")
    wid = sid * NC + cid

    def chunk_body(k, carry):
        base = wid * PER_W + k * CHUNK
        pltpu.sync_copy(i0_hbm.at[pl.ds(base, CHUNK)], i0_v)
        pltpu.sync_copy(i1_hbm.at[pl.ds(base, CHUNK)], i1_v)
        cp0 = pltpu.async_copy(xs_hbm.at[i0_v], rs_v, sem0)
        cp1 = pltpu.async_copy(xd_hbm.at[i1_v], rd_v, sem1)
        cp0.wait()
        cp1.wait()

        lanes = lax.iota(jnp.int32, L)

        def group_body(g, gcarry):
            e0 = g * L
            res = jnp.zeros((L,), jnp.float32)
            for u in range(L):
                e = e0 + u
                acc0 = rs_v[e, pl.ds(0, L)] * rd_v[e, pl.ds(0, L)]
                acc1 = rs_v[e, pl.ds(L, L)] * rd_v[e, pl.ds(L, L)]
                for k in range(2, D // L, 2):
                    acc0 = acc0 + rs_v[e, pl.ds(k * L, L)] * rd_v[e, pl.ds(k * L, L)]
                    acc1 = acc1 + rs_v[e, pl.ds((k + 1) * L, L)] * rd_v[e, pl.ds((k + 1) * L, L)]
                s = jnp.sum(acc0 + acc1)
                res = jnp.where(lanes == u, s, res)
            o_v[pl.ds(e0, L)] = res
            return gcarry

        lax.fori_loop(0, CHUNK // L, group_body, 0)
        pltpu.sync_copy(o_v, out_hbm.at[pl.ds(base, CHUNK)])
        return carry

    lax.fori_loop(0, N_CHUNKS, chunk_body, 0)


@jax.jit
def _run(x_src, x_dst, i0, i1):
    mesh = plsc.VectorSubcoreMesh(core_axis_name="c", subcore_axis_name="s")
    f = pl.kernel(
        _body,
        out_type=jax.ShapeDtypeStruct((EP,), jnp.float32),
        mesh=mesh,
        scratch_types=[
            pltpu.VMEM((CHUNK,), jnp.int32),
            pltpu.VMEM((CHUNK,), jnp.int32),
            pltpu.VMEM((CHUNK, D), jnp.float32),
            pltpu.VMEM((CHUNK, D), jnp.float32),
            pltpu.VMEM((CHUNK,), jnp.float32),
            pltpu.SemaphoreType.DMA,
            pltpu.SemaphoreType.DMA,
        ],
        compiler_params=pltpu.CompilerParams(needs_layout_passes=False),
    )
    return f(x_src, x_dst, i0, i1)


def kernel(x_src, x_dst, edge_label_index):
    pad = EP - E
    idx = jnp.pad(edge_label_index, ((0, 0), (0, pad)))
    out = _run(x_src, x_dst, idx[0], idx[1])
    return out[:E]
